# baseline (speedup 1.0000x reference)
import functools

import jax
import jax.numpy as jnp
from jax import lax
from jax.experimental import pallas as pl
from jax.experimental.pallas import tpu as pltpu
from jax.experimental.pallas import tpu_sc as plsc

B = 16384
D = 64

_info = plsc.get_sparse_core_info()
_NC, _NS, _L = _info.num_cores, _info.num_subcores, _info.num_lanes
NW = _NC * _NS
BPW = B // NW
CH = 128
NCH = BPW // CH
G = _L
NG = BPW // G

_mesh = plsc.VectorSubcoreMesh(core_axis_name="c", subcore_axis_name="s")


@functools.partial(
    pl.kernel,
    out_type=jax.ShapeDtypeStruct((B,), jnp.float32),
    mesh=_mesh,
    compiler_params=pltpu.CompilerParams(
        needs_layout_passes=False, use_tc_tiling_on_sc=False),
    scratch_types=[
        pltpu.VMEM((BPW,), jnp.int32),
        pltpu.VMEM((BPW,), jnp.int32),
        pltpu.VMEM((BPW, D), jnp.float32),
        pltpu.VMEM((BPW, D), jnp.float32),
        pltpu.VMEM((BPW,), jnp.int32),
        pltpu.VMEM((BPW,), jnp.int32),
        pltpu.VMEM((BPW, _L), jnp.float32),
        pltpu.VMEM((BPW, _L), jnp.float32),
        pltpu.VMEM((_L,), jnp.float32),
        pltpu.VMEM((G, _L), jnp.float32),
        pltpu.VMEM((BPW,), jnp.float32),
        pltpu.SemaphoreType.DMA,
    ],
)
def _mf_kernel(uidx_hbm, iidx_hbm, ut_hbm, it_hbm, ub_hbm, ib_hbm, gb_hbm,
               out_hbm,
               uidx_v, iidx_v, urows_v, irows_v, udiv_v, idiv_v, ub_v, ib_v,
               gb_v, m_v, out_v, sem):
    wid = lax.axis_index("s") * _NC + lax.axis_index("c")
    base = wid * BPW

    pltpu.sync_copy(uidx_hbm.at[pl.ds(base, BPW)], uidx_v)
    pltpu.sync_copy(iidx_hbm.at[pl.ds(base, BPW)], iidx_v)
    pltpu.sync_copy(gb_hbm, gb_v.at[pl.ds(0, 1)])

    def bias_rows(i, carry):
        s = pl.ds(i * _L, _L)
        udiv_v[s] = uidx_v[s] >> 4
        idiv_v[s] = iidx_v[s] >> 4
        return carry

    lax.fori_loop(0, BPW // _L, bias_rows, 0)

    copies = []
    for j in range(NCH):
        sl = pl.ds(j * CH, CH)
        copies.append(pltpu.async_copy(ut_hbm.at[uidx_v.at[sl]], urows_v.at[sl], sem))
        copies.append(pltpu.async_copy(it_hbm.at[iidx_v.at[sl]], irows_v.at[sl], sem))
        copies.append(pltpu.async_copy(ub_hbm.at[udiv_v.at[sl]], ub_v.at[sl], sem))
        copies.append(pltpu.async_copy(ib_hbm.at[idiv_v.at[sl]], ib_v.at[sl], sem))
    for cp in copies:
        cp.wait()

    gb = gb_v[pl.ds(0, _L)][0]
    lanes = lax.iota(jnp.int32, _L)
    zeros = jnp.zeros((_L,), jnp.int32)

    def group(g, carry):
        r0 = g * G
        for rr in range(G):
            r = r0 + rr
            acc = urows_v[r, pl.ds(0, _L)] * irows_v[r, pl.ds(0, _L)]
            for c in range(1, D // _L):
                acc = acc + (urows_v[r, pl.ds(c * _L, _L)]
                             * irows_v[r, pl.ds(c * _L, _L)])
            m_v[rr] = acc
        tot = plsc.load_gather(m_v, [lanes, zeros])
        for col in range(1, G):
            tot = tot + plsc.load_gather(
                m_v, [lanes, jnp.full((_L,), col, jnp.int32)])
        iu = uidx_v[pl.ds(r0, _L)]
        ii = iidx_v[pl.ds(r0, _L)]
        ubv = plsc.load_gather(ub_v, [r0 + lanes, iu & 15])
        ibv = plsc.load_gather(ib_v, [r0 + lanes, ii & 15])
        x = tot + ubv + ibv + gb
        pred = 4.0 / (1.0 + jnp.exp(-x)) + 1.0
        out_v[pl.ds(r0, G)] = pred
        return carry

    lax.fori_loop(0, NG, group, 0)
    pltpu.sync_copy(out_v, out_hbm.at[pl.ds(base, BPW)])


def kernel(user_indices, item_indices, user_table, item_table, user_bias,
           item_bias, global_bias):
    ub2 = user_bias.reshape(user_bias.shape[0] // _L, _L)
    ib2 = item_bias.reshape(item_bias.shape[0] // _L, _L)
    return _mf_kernel(user_indices.astype(jnp.int32),
                      item_indices.astype(jnp.int32),
                      user_table, item_table, ub2, ib2, global_bias)


# speedup vs baseline: 1.8818x; 1.8818x over previous
import functools

import jax
import jax.numpy as jnp
from jax import lax
from jax.experimental import pallas as pl
from jax.experimental.pallas import tpu as pltpu
from jax.experimental.pallas import tpu_sc as plsc

B = 16384
D = 64

_info = plsc.get_sparse_core_info()
_NC, _NS, _L = _info.num_cores, _info.num_subcores, _info.num_lanes
NW = _NC * _NS
BPW = B // NW
CH = 32
NCH = BPW // CH
G = _L

_mesh = plsc.VectorSubcoreMesh(core_axis_name="c", subcore_axis_name="s")
_params = pltpu.CompilerParams(needs_layout_passes=False)


@functools.partial(
    pl.kernel,
    out_type=jax.ShapeDtypeStruct((B,), jnp.float32),
    mesh=_mesh,
    compiler_params=_params,
    scratch_types=[
        pltpu.VMEM((BPW,), jnp.int32),
        pltpu.VMEM((BPW,), jnp.int32),
        pltpu.VMEM((CH, 8, D), jnp.float32),
        pltpu.VMEM((CH, 8, D), jnp.float32),
        pltpu.VMEM((BPW,), jnp.float32),
        pltpu.SemaphoreType.DMA,
    ],
)
def _dot_kernel(uidx_hbm, iidx_hbm, ut3_hbm, it3_hbm, dots_hbm,
                uidx_v, iidx_v, utile_v, itile_v, dots_v,
                sem):
    wid = lax.axis_index("s") * _NC + lax.axis_index("c")
    base = wid * BPW

    pltpu.sync_copy(uidx_hbm.at[pl.ds(base, BPW)], uidx_v)
    pltpu.sync_copy(iidx_hbm.at[pl.ds(base, BPW)], iidx_v)
    lanes = lax.iota(jnp.int32, _L)

    def chunk(g, carry):
        copies = []
        for gg in range(CH // G):
            tu = uidx_v[pl.ds(g * CH + gg * G, G)] >> 3
            ti = iidx_v[pl.ds(g * CH + gg * G, G)] >> 3
            for k in range(G):
                slot = gg * G + k
                copies.append(
                    pltpu.async_copy(ut3_hbm.at[tu[k]], utile_v.at[slot], sem))
                copies.append(
                    pltpu.async_copy(it3_hbm.at[ti[k]], itile_v.at[slot], sem))
        for cp in copies:
            cp.wait()
        for gg in range(CH // G):
            ro = g * CH + gg * G
            iu = uidx_v[pl.ds(ro, G)]
            ii = iidx_v[pl.ds(ro, G)]
            su = iu & 7
            si = ii & 7
            rows = gg * G + lanes
            acc = None
            for d in range(D):
                col = jnp.full((_L,), d, jnp.int32)
                ud = plsc.load_gather(utile_v, [rows, su, col])
                vd = plsc.load_gather(itile_v, [rows, si, col])
                acc = ud * vd if acc is None else acc + ud * vd
            dots_v[pl.ds(ro, G)] = acc
        return carry

    lax.fori_loop(0, NCH, chunk, 0)
    pltpu.sync_copy(dots_v, dots_hbm.at[pl.ds(base, BPW)])


@functools.partial(
    pl.kernel,
    out_type=jax.ShapeDtypeStruct((B,), jnp.float32),
    mesh=_mesh,
    compiler_params=pltpu.CompilerParams(
        needs_layout_passes=False, use_tc_tiling_on_sc=False),
    scratch_types=[
        pltpu.VMEM((BPW,), jnp.int32),
        pltpu.VMEM((BPW,), jnp.int32),
        pltpu.VMEM((BPW,), jnp.int32),
        pltpu.VMEM((BPW,), jnp.int32),
        pltpu.VMEM((BPW, D), jnp.float32),
        pltpu.VMEM((BPW, 32), jnp.float32),
        pltpu.VMEM((_L,), jnp.float32),
        pltpu.VMEM((BPW,), jnp.float32),
        pltpu.VMEM((BPW,), jnp.float32),
        pltpu.SemaphoreType.DMA,
    ],
)
def _bias_kernel(uidx_hbm, iidx_hbm, ub2_hbm, ib2_hbm, gb_hbm, dots_hbm,
                 out_hbm,
                 uidx_v, iidx_v, ubr_v, ibr_v, ub_v, ib_v, gb_v, dots_v,
                 out_v, sem):
    wid = lax.axis_index("s") * _NC + lax.axis_index("c")
    base = wid * BPW

    pltpu.sync_copy(uidx_hbm.at[pl.ds(base, BPW)], uidx_v)
    pltpu.sync_copy(iidx_hbm.at[pl.ds(base, BPW)], iidx_v)
    pltpu.sync_copy(dots_hbm.at[pl.ds(base, BPW)], dots_v)
    pltpu.sync_copy(gb_hbm, gb_v.at[pl.ds(0, 1)])

    def bias_rows(i, carry):
        s = pl.ds(i * _L, _L)
        ubr_v[s] = uidx_v[s] >> 6
        ibr_v[s] = iidx_v[s] >> 5
        return carry

    lax.fori_loop(0, BPW // _L, bias_rows, 0)

    copies = []
    for j in range(4):
        sl = pl.ds(j * 128, 128)
        copies.append(pltpu.async_copy(ub2_hbm.at[ubr_v.at[sl]], ub_v.at[sl], sem))
        copies.append(pltpu.async_copy(ib2_hbm.at[ibr_v.at[sl]], ib_v.at[sl], sem))
    for cp in copies:
        cp.wait()

    gb = gb_v[pl.ds(0, _L)][0]
    lanes = lax.iota(jnp.int32, _L)

    def group(g, carry):
        r0 = g * G
        iu = uidx_v[pl.ds(r0, G)]
        ii = iidx_v[pl.ds(r0, G)]
        ubv = plsc.load_gather(ub_v, [r0 + lanes, iu & 63])
        ibv = plsc.load_gather(ib_v, [r0 + lanes, ii & 31])
        x = dots_v[pl.ds(r0, G)] + ubv + ibv + gb
        out_v[pl.ds(r0, G)] = 4.0 / (1.0 + jnp.exp(-x)) + 1.0
        return carry

    lax.fori_loop(0, BPW // G, group, 0)
    pltpu.sync_copy(out_v, out_hbm.at[pl.ds(base, BPW)])


def kernel(user_indices, item_indices, user_table, item_table, user_bias,
           item_bias, global_bias):
    ui = user_indices.astype(jnp.int32)
    ii = item_indices.astype(jnp.int32)
    ut3 = user_table.reshape(-1, 8, D)
    it3 = item_table.reshape(-1, 8, D)
    ub2 = user_bias.reshape(-1, 64)
    ib2 = item_bias.reshape(-1, 32)
    dots = _dot_kernel(ui, ii, ut3, it3)
    return _bias_kernel(ui, ii, ub2, ib2, global_bias, dots)
